# baseline (speedup 1.0000x reference)
import jax
import jax.numpy as jnp
from jax import lax
from jax.experimental import pallas as pl
from jax.experimental.pallas import tpu as pltpu

_Q = 4
_K = 1024
_D = 128
_BN = 1024


def _rvq_kernel(x_ref, cb_ref, xq_ref, loss_ref, i0_ref, i1_ref, i2_ref, i3_ref):
    idx_refs = (i0_ref, i1_ref, i2_ref, i3_ref)
    res = x_ref[...]
    xq = jnp.zeros_like(res)
    sse = jnp.zeros((1, 1), jnp.float32)
    lane_iota = lax.broadcasted_iota(jnp.int32, (_BN, _K), 1)
    for q in range(_Q):
        cb = cb_ref[q]
        c2 = jnp.sum(cb * cb, axis=1)
        z2 = jnp.sum(res * res, axis=1, keepdims=True)
        s = lax.dot_general(res, cb, (((1,), (1,)), ((), ())),
                            preferred_element_type=jnp.float32)
        d = (z2 - 2.0 * s) + c2[None, :]
        m = jnp.min(d, axis=1, keepdims=True)
        idx = jnp.min(jnp.where(d == m, lane_iota, _K), axis=1)
        onehot = (lane_iota == idx[:, None]).astype(jnp.float32)
        zq = lax.dot_general(onehot, cb, (((1,), (0,)), ((), ())),
                             preferred_element_type=jnp.float32,
                             precision=lax.Precision.HIGHEST)
        res = res - zq
        xq = xq + zq
        sse = sse + jnp.sum(res * res, keepdims=True).reshape(1, 1)
        idx_refs[q][...] = idx[:, None]
    xq_ref[...] = xq
    loss_ref[...] = jnp.broadcast_to(sse[None], (1, 1, 128))


def kernel(x, codebooks):
    n = x.shape[0]
    nblk = n // _BN
    out_shape = (
        jax.ShapeDtypeStruct((n, _D), jnp.float32),
        jax.ShapeDtypeStruct((nblk, 1, 128), jnp.float32),
        jax.ShapeDtypeStruct((n, 1), jnp.int32),
        jax.ShapeDtypeStruct((n, 1), jnp.int32),
        jax.ShapeDtypeStruct((n, 1), jnp.int32),
        jax.ShapeDtypeStruct((n, 1), jnp.int32),
    )
    xq, losses, i0, i1, i2, i3 = pl.pallas_call(
        _rvq_kernel,
        grid=(nblk,),
        in_specs=[
            pl.BlockSpec((_BN, _D), lambda i: (i, 0)),
            pl.BlockSpec((_Q, _K, _D), lambda i: (0, 0, 0)),
        ],
        out_specs=[
            pl.BlockSpec((_BN, _D), lambda i: (i, 0)),
            pl.BlockSpec((1, 1, 128), lambda i: (i, 0, 0)),
            pl.BlockSpec((_BN, 1), lambda i: (i, 0)),
            pl.BlockSpec((_BN, 1), lambda i: (i, 0)),
            pl.BlockSpec((_BN, 1), lambda i: (i, 0)),
            pl.BlockSpec((_BN, 1), lambda i: (i, 0)),
        ],
        out_shape=out_shape,
        compiler_params=pltpu.CompilerParams(
            dimension_semantics=("arbitrary",)),
    )(x, codebooks)
    indices = jnp.concatenate([i0, i1, i2, i3], axis=1)
    loss = jnp.sum(losses[:, 0, 0]) * (1.25 / (_Q * n * _D))
    return xq, loss, indices


# speedup vs baseline: 2.0012x; 2.0012x over previous
import jax
import jax.numpy as jnp
from jax import lax
from jax.experimental import pallas as pl
from jax.experimental.pallas import tpu as pltpu

_Q = 4
_K = 1024
_D = 128
_BN = 1024


def _rvq_kernel(x_ref, cb_ref, xq_ref, loss_ref, i0_ref, i1_ref, i2_ref, i3_ref):
    idx_refs = (i0_ref, i1_ref, i2_ref, i3_ref)
    res = x_ref[...]
    xq = jnp.zeros_like(res)
    sse = jnp.zeros((1, 1), jnp.float32)
    lane_iota = lax.broadcasted_iota(jnp.int32, (_BN, _K), 1)
    for q in range(_Q):
        cb = cb_ref[q]
        c2 = jnp.sum(cb * cb, axis=1)
        z2 = jnp.sum(res * res, axis=1, keepdims=True)
        s = lax.dot_general(res, cb, (((1,), (1,)), ((), ())),
                            preferred_element_type=jnp.float32)
        d = (z2 - 2.0 * s) + c2[None, :]
        m = jnp.min(d, axis=1, keepdims=True)
        idx = jnp.min(jnp.where(d == m, lane_iota, _K), axis=1)
        onehot = (lane_iota == idx[:, None]).astype(jnp.float32)
        cb_hi = cb.astype(jnp.bfloat16).astype(jnp.float32)
        cb_lo = cb - cb_hi
        zq = (lax.dot_general(onehot, cb_hi, (((1,), (0,)), ((), ())),
                              preferred_element_type=jnp.float32)
              + lax.dot_general(onehot, cb_lo, (((1,), (0,)), ((), ())),
                                preferred_element_type=jnp.float32))
        res = res - zq
        xq = xq + zq
        sse = sse + jnp.sum(res * res, keepdims=True).reshape(1, 1)
        idx_refs[q][...] = idx[:, None]
    xq_ref[...] = xq
    loss_ref[...] = jnp.broadcast_to(sse[None], (1, 1, 128))


def kernel(x, codebooks):
    n = x.shape[0]
    nblk = n // _BN
    out_shape = (
        jax.ShapeDtypeStruct((n, _D), jnp.float32),
        jax.ShapeDtypeStruct((nblk, 1, 128), jnp.float32),
        jax.ShapeDtypeStruct((n, 1), jnp.int32),
        jax.ShapeDtypeStruct((n, 1), jnp.int32),
        jax.ShapeDtypeStruct((n, 1), jnp.int32),
        jax.ShapeDtypeStruct((n, 1), jnp.int32),
    )
    xq, losses, i0, i1, i2, i3 = pl.pallas_call(
        _rvq_kernel,
        grid=(nblk,),
        in_specs=[
            pl.BlockSpec((_BN, _D), lambda i: (i, 0)),
            pl.BlockSpec((_Q, _K, _D), lambda i: (0, 0, 0)),
        ],
        out_specs=[
            pl.BlockSpec((_BN, _D), lambda i: (i, 0)),
            pl.BlockSpec((1, 1, 128), lambda i: (i, 0, 0)),
            pl.BlockSpec((_BN, 1), lambda i: (i, 0)),
            pl.BlockSpec((_BN, 1), lambda i: (i, 0)),
            pl.BlockSpec((_BN, 1), lambda i: (i, 0)),
            pl.BlockSpec((_BN, 1), lambda i: (i, 0)),
        ],
        out_shape=out_shape,
        compiler_params=pltpu.CompilerParams(
            dimension_semantics=("parallel",)),
    )(x, codebooks)
    indices = jnp.concatenate([i0, i1, i2, i3], axis=1)
    loss = jnp.sum(losses[:, 0, 0]) * (1.25 / (_Q * n * _D))
    return xq, loss, indices


# speedup vs baseline: 2.8411x; 1.4197x over previous
import jax
import jax.numpy as jnp
from jax import lax
from jax.experimental import pallas as pl
from jax.experimental.pallas import tpu as pltpu

_Q = 4
_K = 1024
_D = 128
_BN = 1024


def _rvq_kernel(x_ref, cb_ref, xq_ref, loss_ref,
                i0_ref, i1_ref, i2_ref, i3_ref,
                c2_ref, cbd_ref, cbg_ref):
    idx_refs = (i0_ref, i1_ref, i2_ref, i3_ref)

    @pl.when(pl.program_id(0) == 0)
    def _init_codebook_scratch():
        for q in range(_Q):
            cb = cb_ref[q]
            c2_ref[q] = jnp.sum(cb * cb, axis=1)[None, :]
            hi = cb.astype(jnp.bfloat16)
            lo = (cb - hi.astype(jnp.float32)).astype(jnp.bfloat16)
            cbd_ref[q] = hi
            cbg_ref[q] = jnp.concatenate([hi, lo], axis=1)

    res = x_ref[...]
    xq = jnp.zeros_like(res)
    sse = jnp.zeros((1, 1), jnp.float32)
    liota = lax.broadcasted_iota(jnp.int32, (_BN, _K), 1).astype(jnp.float32)
    for q in range(_Q):
        z2 = jnp.sum(res * res, axis=1, keepdims=True)
        nres = (res * -2.0).astype(jnp.bfloat16)
        s = lax.dot_general(nres, cbd_ref[q], (((1,), (1,)), ((), ())),
                            preferred_element_type=jnp.float32)
        d = (z2 + s) + c2_ref[q]
        m = jnp.min(d, axis=1, keepdims=True)
        idxf = jnp.min(jnp.where(d == m, liota, 2048.0), axis=1)
        onehot = (liota == idxf[:, None]).astype(jnp.bfloat16)
        zq2 = lax.dot_general(onehot, cbg_ref[q], (((1,), (0,)), ((), ())),
                              preferred_element_type=jnp.float32)
        zq = zq2[:, :_D] + zq2[:, _D:]
        res = res - zq
        xq = xq + zq
        sse = sse + jnp.sum(res * res, keepdims=True).reshape(1, 1)
        idx_refs[q][...] = idxf.astype(jnp.int32)[:, None]
    xq_ref[...] = xq
    loss_ref[...] = jnp.broadcast_to(sse[None], (1, 1, 128))


def kernel(x, codebooks):
    n = x.shape[0]
    nblk = n // _BN
    out_shape = (
        jax.ShapeDtypeStruct((n, _D), jnp.float32),
        jax.ShapeDtypeStruct((nblk, 1, 128), jnp.float32),
        jax.ShapeDtypeStruct((n, 1), jnp.int32),
        jax.ShapeDtypeStruct((n, 1), jnp.int32),
        jax.ShapeDtypeStruct((n, 1), jnp.int32),
        jax.ShapeDtypeStruct((n, 1), jnp.int32),
    )
    xq, losses, i0, i1, i2, i3 = pl.pallas_call(
        _rvq_kernel,
        grid=(nblk,),
        in_specs=[
            pl.BlockSpec((_BN, _D), lambda i: (i, 0)),
            pl.BlockSpec((_Q, _K, _D), lambda i: (0, 0, 0)),
        ],
        out_specs=[
            pl.BlockSpec((_BN, _D), lambda i: (i, 0)),
            pl.BlockSpec((1, 1, 128), lambda i: (i, 0, 0)),
            pl.BlockSpec((_BN, 1), lambda i: (i, 0)),
            pl.BlockSpec((_BN, 1), lambda i: (i, 0)),
            pl.BlockSpec((_BN, 1), lambda i: (i, 0)),
            pl.BlockSpec((_BN, 1), lambda i: (i, 0)),
        ],
        out_shape=out_shape,
        scratch_shapes=[
            pltpu.VMEM((_Q, 1, _K), jnp.float32),
            pltpu.VMEM((_Q, _K, _D), jnp.bfloat16),
            pltpu.VMEM((_Q, _K, 2 * _D), jnp.bfloat16),
        ],
        compiler_params=pltpu.CompilerParams(
            dimension_semantics=("arbitrary",)),
    )(x, codebooks)
    indices = jnp.concatenate([i0, i1, i2, i3], axis=1)
    loss = jnp.sum(losses[:, 0, 0]) * (1.25 / (_Q * n * _D))
    return xq, loss, indices


# speedup vs baseline: 3.7986x; 1.3370x over previous
import jax
import jax.numpy as jnp
from jax import lax
from jax.experimental import pallas as pl
from jax.experimental.pallas import tpu as pltpu

_Q = 4
_K = 1024
_D = 128
_BN = 1024


def _rvq_kernel(x_ref, cb_ref, xq_ref, loss_ref,
                i0_ref, i1_ref, i2_ref, i3_ref,
                c2_ref, cbd_ref, cbg_ref):
    idx_refs = (i0_ref, i1_ref, i2_ref, i3_ref)

    @pl.when(pl.program_id(0) == 0)
    def _init_codebook_scratch():
        for q in range(_Q):
            cb = cb_ref[q]
            c2_ref[q] = jnp.sum(cb * cb, axis=1)[None, :]
            hi = cb.astype(jnp.bfloat16)
            lo = (cb - hi.astype(jnp.float32)).astype(jnp.bfloat16)
            cbd_ref[q] = hi
            cbg_ref[q] = jnp.concatenate([hi, lo], axis=1)

    _H = _BN // 2
    halves = [x_ref[0:_H, :], x_ref[_H:_BN, :]]
    xqs = [jnp.zeros((_H, _D), jnp.float32) for _ in range(2)]
    sse = jnp.zeros((1, 1), jnp.float32)
    liota = lax.broadcasted_iota(jnp.int32, (_H, _K), 1).astype(jnp.float32)
    for q in range(_Q):
        idxs = [None, None]
        for h in range(2):
            res = halves[h]
            z2 = jnp.sum(res * res, axis=1, keepdims=True)
            nres = (res * -2.0).astype(jnp.bfloat16)
            s = lax.dot_general(nres, cbd_ref[q], (((1,), (1,)), ((), ())),
                                preferred_element_type=jnp.float32)
            d = (z2 + s) + c2_ref[q]
            m = jnp.min(d, axis=1, keepdims=True)
            idxf = jnp.min(jnp.where(d == m, liota, 2048.0), axis=1)
            onehot = (liota == idxf[:, None]).astype(jnp.bfloat16)
            zq2 = lax.dot_general(onehot, cbg_ref[q], (((1,), (0,)), ((), ())),
                                  preferred_element_type=jnp.float32)
            zq = zq2[:, :_D] + zq2[:, _D:]
            res = res - zq
            halves[h] = res
            xqs[h] = xqs[h] + zq
            sse = sse + jnp.sum(res * res, keepdims=True).reshape(1, 1)
            idxs[h] = idxf.astype(jnp.int32)[:, None]
        idx_refs[q][0:_H, :] = idxs[0]
        idx_refs[q][_H:_BN, :] = idxs[1]
    xq_ref[0:_H, :] = xqs[0]
    xq_ref[_H:_BN, :] = xqs[1]
    loss_ref[...] = jnp.broadcast_to(sse[None], (1, 1, 128))


def kernel(x, codebooks):
    n = x.shape[0]
    nblk = n // _BN
    out_shape = (
        jax.ShapeDtypeStruct((n, _D), jnp.float32),
        jax.ShapeDtypeStruct((nblk, 1, 128), jnp.float32),
        jax.ShapeDtypeStruct((n, 1), jnp.int32),
        jax.ShapeDtypeStruct((n, 1), jnp.int32),
        jax.ShapeDtypeStruct((n, 1), jnp.int32),
        jax.ShapeDtypeStruct((n, 1), jnp.int32),
    )
    xq, losses, i0, i1, i2, i3 = pl.pallas_call(
        _rvq_kernel,
        grid=(nblk,),
        in_specs=[
            pl.BlockSpec((_BN, _D), lambda i: (i, 0)),
            pl.BlockSpec((_Q, _K, _D), lambda i: (0, 0, 0)),
        ],
        out_specs=[
            pl.BlockSpec((_BN, _D), lambda i: (i, 0)),
            pl.BlockSpec((1, 1, 128), lambda i: (i, 0, 0)),
            pl.BlockSpec((_BN, 1), lambda i: (i, 0)),
            pl.BlockSpec((_BN, 1), lambda i: (i, 0)),
            pl.BlockSpec((_BN, 1), lambda i: (i, 0)),
            pl.BlockSpec((_BN, 1), lambda i: (i, 0)),
        ],
        out_shape=out_shape,
        scratch_shapes=[
            pltpu.VMEM((_Q, 1, _K), jnp.float32),
            pltpu.VMEM((_Q, _K, _D), jnp.bfloat16),
            pltpu.VMEM((_Q, _K, 2 * _D), jnp.bfloat16),
        ],
        compiler_params=pltpu.CompilerParams(
            dimension_semantics=("arbitrary",)),
    )(x, codebooks)
    indices = jnp.concatenate([i0, i1, i2, i3], axis=1)
    loss = jnp.sum(losses[:, 0, 0]) * (1.25 / (_Q * n * _D))
    return xq, loss, indices
